# baseline (device time: 732245 ns/iter reference)
import jax
import jax.numpy as jnp
from jax import lax
from jax.experimental import pallas as pl
from jax.experimental.pallas import tpu as pltpu

N_DEV = 16
M = 4096
K = 256
N = 8192
HN = N // 2
MC = M // N_DEV
NSUB = 4
RH = MC // NSUB
NSLOT = 4


def _ar_body(x_ref, w_ref, out_hbm, send_buf, recv_buf,
             out_sems, send_sems, recv_sems):
    p = lax.axis_index("i")
    left = lax.rem(p + N_DEV - 1, N_DEV)
    right = lax.rem(p + 1, N_DEV)

    dst = (right, left)

    def rs_chunk(d, s):
        if d == 0:
            return lax.rem(p + N_DEV - s - 1, N_DEV)
        return lax.rem(p + s + 1, N_DEV)

    def ag_chunk(d, t):
        if d == 0:
            return lax.rem(p + N_DEV - t, N_DEV)
        return lax.rem(p + t, N_DEV)

    def partial(c, d, r):
        xs = x_ref[pl.ds(c * MC + r * RH, RH), :]
        ws = w_ref[:, d * HN:(d + 1) * HN]
        return jnp.dot(xs, ws, preferred_element_type=jnp.float32)

    def out_slice(c, d, r):
        return out_hbm.at[pl.ds(c * MC + r * RH, RH), pl.ds(d * HN, HN)]

    def rs_rdma(s, r, d):
        slot = s % NSLOT
        return pltpu.make_async_remote_copy(
            src_ref=send_buf.at[d, pl.ds(r * RH, RH)],
            dst_ref=recv_buf.at[d, slot, pl.ds(r * RH, RH)],
            send_sem=send_sems.at[d, slot, r],
            recv_sem=recv_sems.at[d, slot, r],
            device_id=(dst[d],),
            device_id_type=pl.DeviceIdType.MESH,
        )

    def ag_rdma(t, r, d):
        slot = (N_DEV - 1 + t) % NSLOT
        src = send_buf.at[d, pl.ds(r * RH, RH)] if t == 0 else \
            recv_buf.at[d, (N_DEV - 2 + t) % NSLOT, pl.ds(r * RH, RH)]
        return pltpu.make_async_remote_copy(
            src_ref=src,
            dst_ref=recv_buf.at[d, slot, pl.ds(r * RH, RH)],
            send_sem=send_sems.at[d, slot, r],
            recv_sem=recv_sems.at[d, slot, r],
            device_id=(dst[d],),
            device_id_type=pl.DeviceIdType.MESH,
        )

    barrier_sem = pltpu.get_barrier_semaphore()
    for nbr in (left, right):
        pl.semaphore_signal(barrier_sem, inc=1, device_id=(nbr,),
                            device_id_type=pl.DeviceIdType.MESH)
    pl.semaphore_wait(barrier_sem, 2)

    for d in (0, 1):
        for r in range(NSUB):
            send_buf[d, r * RH:(r + 1) * RH, :] = \
                partial(p, d, r).astype(jnp.bfloat16)
    inflight = {}
    for r in range(NSUB):
        for d in (0, 1):
            rd = rs_rdma(0, r, d)
            rd.start()
            inflight[(r, d)] = rd

    out_pend = {}

    def issue_store(src_ref, c, d, r, phase):
        key = (d, phase % 2, r)
        if key in out_pend:
            out_pend.pop(key).wait()
        cp = pltpu.make_async_copy(src_ref, out_slice(c, d, r),
                                   out_sems.at[d, phase % 2, r])
        cp.start()
        out_pend[key] = cp

    for s in range(N_DEV - 1):
        slot = s % NSLOT
        parts = {(r, d): partial(rs_chunk(d, s), d, r)
                 for r in range(NSUB) for d in (0, 1)}
        for r in range(NSUB):
            for d in (0, 1):
                rows = pl.ds(r * RH, RH)
                inflight.pop((r, d)).wait()
                acc = (recv_buf[d, slot, r * RH:(r + 1) * RH, :]
                       .astype(jnp.float32) + parts[(r, d)])
                if s < N_DEV - 2:
                    send_buf[d, r * RH:(r + 1) * RH, :] = \
                        acc.astype(jnp.bfloat16)
                    rd = rs_rdma(s + 1, r, d)
                    rd.start()
                    inflight[(r, d)] = rd
                else:
                    y = (acc * jax.nn.sigmoid(acc)).astype(jnp.bfloat16)
                    send_buf[d, r * RH:(r + 1) * RH, :] = y
                    rd = ag_rdma(0, r, d)
                    rd.start()
                    inflight[(r, d)] = rd
                    own = rs_chunk(d, N_DEV - 2)
                    issue_store(send_buf.at[d, rows], own, d, r, phase=1)

    for t in range(N_DEV - 1):
        slot = (N_DEV - 1 + t) % NSLOT
        for r in range(NSUB):
            for d in (0, 1):
                rows = pl.ds(r * RH, RH)
                inflight.pop((r, d)).wait()
                if t < N_DEV - 2:
                    rd = ag_rdma(t + 1, r, d)
                    rd.start()
                    inflight[(r, d)] = rd
                issue_store(recv_buf.at[d, slot, rows], ag_chunk(d, t),
                            d, r, phase=t)

    for cp in out_pend.values():
        cp.wait()


def kernel(x, w_mat):
    return pl.pallas_call(
        _ar_body,
        out_shape=jax.ShapeDtypeStruct((M, N), jnp.bfloat16),
        in_specs=[
            pl.BlockSpec(memory_space=pltpu.VMEM),
            pl.BlockSpec(memory_space=pltpu.VMEM),
        ],
        out_specs=pl.BlockSpec(memory_space=pl.ANY),
        scratch_shapes=[
            pltpu.VMEM((2, MC, HN), jnp.bfloat16),
            pltpu.VMEM((2, NSLOT, MC, HN), jnp.bfloat16),
            pltpu.SemaphoreType.DMA((2, 2, NSUB)),
            pltpu.SemaphoreType.DMA((2, NSLOT, NSUB)),
            pltpu.SemaphoreType.DMA((2, NSLOT, NSUB)),
        ],
        compiler_params=pltpu.CompilerParams(
            collective_id=0, vmem_limit_bytes=60 * 1024 * 1024),
    )(x, w_mat)
